# Initial kernel scaffold; baseline (speedup 1.0000x reference)
import jax
import jax.numpy as jnp
from jax.experimental import pallas as pl


def kernel(local_x, global_x, local_edge_index, global_edge_index, local_edge_weight, global_edge_weight, readout_batch, local_params, global_params):
    raise NotImplementedError("write your pallas kernel here")


# speedup vs baseline: 22.4371x; 22.4371x over previous
import jax
import jax.numpy as jnp
from jax import lax
from jax.experimental import pallas as pl
from jax.experimental.pallas import tpu as pltpu
from jax.experimental.pallas import tpu_sc as plsc

N = 10000
NP = 10240
F = 128
T = 4
EL_PAD = 327680
EG_PAD = 163840
NBLK_L = 10
NBLK_G = 5
SLICE = NP // 16

_f32 = jnp.float32
_i32 = jnp.int32


def _mesh():
    return plsc.VectorSubcoreMesh(core_axis_name="c", subcore_axis_name="s")


def _babylon_dinv(x):
    s = 0.5 * (1.0 + x)
    for _ in range(18):
        s = 0.5 * (s + x / s)
    return 1.0 / s


def _deg_body(dstl, wl, dstg, wg, dinvl, dinvg, buf_v, dst_b, w_b, spdeg):
    c = lax.axis_index("c")
    s = lax.axis_index("s")
    zero16 = jnp.zeros((16,), _f32)

    def run_graph(dst_h, w_h, out_h, nblk):
        def zbody(i, carry):
            buf_v[pl.ds(i * 16, 16)] = zero16
            return carry
        lax.fori_loop(0, NP // 16, zbody, 0)

        @pl.when(s == 0)
        def _():
            pltpu.sync_copy(buf_v, spdeg)
        plsc.subcore_barrier()

        base = s * nblk * 16

        def sblk(blk, carry):
            r0 = base + blk * 16
            pltpu.sync_copy(dst_h.at[pl.ds(r0, 16)], dst_b)
            pltpu.sync_copy(w_h.at[pl.ds(r0, 16)], w_b)
            for k in range(16):
                pltpu.sync_copy(w_b.at[k], spdeg.at[dst_b.at[k]], add=True)
            return carry
        lax.fori_loop(0, nblk, sblk, 0)
        plsc.subcore_barrier()

        pltpu.sync_copy(spdeg.at[pl.ds(s * SLICE, SLICE)],
                        buf_v.at[pl.ds(0, SLICE)])

        def dbody(i, carry):
            x = buf_v[pl.ds(i * 16, 16)] + 1.0
            buf_v[pl.ds(i * 16, 16)] = _babylon_dinv(x)
            return carry
        lax.fori_loop(0, SLICE // 16, dbody, 0)
        pltpu.sync_copy(buf_v.at[pl.ds(0, SLICE)],
                        out_h.at[pl.ds(s * SLICE, SLICE)])

    @pl.when(c == 0)
    def _():
        run_graph(dstl, wl, dinvl, NBLK_L)

    @pl.when(c == 1)
    def _():
        run_graph(dstg, wg, dinvg, NBLK_G)


@jax.jit
def _deg_kernel(dstl, wl, dstg, wg):
    out_type = [
        jax.ShapeDtypeStruct((NP,), _f32),
        jax.ShapeDtypeStruct((NP,), _f32),
    ]
    scratch = [
        pltpu.VMEM((NP,), _f32),
        pltpu.VMEM((16, 128), _i32),
        pltpu.VMEM((16, 128), _f32),
        pltpu.VMEM_SHARED((NP,), _f32),
    ]
    return pl.kernel(_deg_body, out_type=out_type, mesh=_mesh(),
                     scratch_types=scratch)(dstl, wl, dstg, wg)


def _prescale_body(x_ref, dinv_ref, y_ref):
    y_ref[...] = x_ref[...] * dinv_ref[...]


@jax.jit
def _prescale(x, dinv):
    blk = 1024
    return pl.pallas_call(
        _prescale_body,
        grid=(2, T, NP // blk),
        in_specs=[
            pl.BlockSpec((1, 1, blk, F), lambda g, t, i: (g, t, i, 0)),
            pl.BlockSpec((1, blk, 1), lambda g, t, i: (g, i, 0)),
        ],
        out_specs=pl.BlockSpec((1, 1, blk, F), lambda g, t, i: (g, t, i, 0)),
        out_shape=jax.ShapeDtypeStruct((2, T, NP, F), _f32),
    )(x, dinv)


def _scatter_pass(y_h, src_h, dst_h, w_h, out_h, nblk, s,
                  gbuf, src_b, dst_b, w_b, accum):
    pltpu.sync_copy(y_h.at[pl.ds(s * SLICE, SLICE)],
                    accum.at[pl.ds(s * SLICE, SLICE)])
    plsc.subcore_barrier()

    base = s * nblk * 16

    def bblk(blk, carry):
        r0 = base + blk * 16
        pltpu.sync_copy(src_h.at[pl.ds(r0, 16)], src_b)
        pltpu.sync_copy(dst_h.at[pl.ds(r0, 16)], dst_b)
        pltpu.sync_copy(w_h.at[pl.ds(r0, 16)], w_b)

        def kbody(k, carry2):
            pltpu.sync_copy(y_h.at[src_b.at[k]], gbuf)

            def gbody(g, carry3):
                wv = w_b[k, pl.ds(g * 16, 16)]
                for i in range(16):
                    wsc = lax.index_in_dim(wv, i, keepdims=False)
                    e = g * 16 + i
                    for j in range(8):
                        gbuf[e, pl.ds(j * 16, 16)] = (
                            gbuf[e, pl.ds(j * 16, 16)] * wsc)
                return carry3
            lax.fori_loop(0, 8, gbody, 0)
            pltpu.sync_copy(gbuf, accum.at[dst_b.at[k]], add=True)
            return carry2
        lax.fori_loop(0, 16, kbody, 0)
        return carry
    lax.fori_loop(0, nblk, bblk, 0)
    plsc.subcore_barrier()

    pltpu.sync_copy(accum.at[pl.ds(s * SLICE, SLICE)],
                    out_h.at[pl.ds(s * SLICE, SLICE)])


def _scatter_body(yl0, yl1, yl2, yl3, yg0, yg1, yg2, yg3,
                  srcl, dstl, wl, srcg, dstg, wg,
                  zl0, zl1, zl2, zl3, zg0, zg1, zg2, zg3,
                  gbuf, src_b, dst_b, w_b, accum):
    c = lax.axis_index("c")
    s = lax.axis_index("s")
    scr = (gbuf, src_b, dst_b, w_b, accum)

    @pl.when(c == 0)
    def _():
        _scatter_pass(yl0, srcl, dstl, wl, zl0, NBLK_L, s, *scr)
        _scatter_pass(yl1, srcl, dstl, wl, zl1, NBLK_L, s, *scr)
        _scatter_pass(yl2, srcl, dstl, wl, zl2, NBLK_L, s, *scr)

    @pl.when(c == 1)
    def _():
        _scatter_pass(yl3, srcl, dstl, wl, zl3, NBLK_L, s, *scr)
        _scatter_pass(yg0, srcg, dstg, wg, zg0, NBLK_G, s, *scr)
        _scatter_pass(yg1, srcg, dstg, wg, zg1, NBLK_G, s, *scr)
        _scatter_pass(yg2, srcg, dstg, wg, zg2, NBLK_G, s, *scr)
        _scatter_pass(yg3, srcg, dstg, wg, zg3, NBLK_G, s, *scr)


@jax.jit
def _scatter_kernel(y, srcl, dstl, wl, srcg, dstg, wg):
    out_type = [jax.ShapeDtypeStruct((NP, F), _f32) for _ in range(8)]
    scratch = [
        pltpu.VMEM((128, 128), _f32),
        pltpu.VMEM((16, 128), _i32),
        pltpu.VMEM((16, 128), _i32),
        pltpu.VMEM((16, 128), _f32),
        pltpu.VMEM_SHARED((NP, F), _f32),
    ]
    return pl.kernel(_scatter_body, out_type=out_type, mesh=_mesh(),
                     scratch_types=scratch)(
        y[0, 0], y[0, 1], y[0, 2], y[0, 3],
        y[1, 0], y[1, 1], y[1, 2], y[1, 3],
        srcl, dstl, wl, srcg, dstg, wg)


def _wprep_body(wcat, l1s, bcat, lbcat, wc, bc):
    for g in range(3):
        l1 = l1s[g]
        wc[:, g * 128:(g + 1) * 128] = jnp.dot(
            wcat[:, g * 128:(g + 1) * 128], l1, preferred_element_type=_f32)
        bc[:, g * 128:(g + 1) * 128] = jnp.dot(
            bcat[:, g * 128:(g + 1) * 128], l1, preferred_element_type=_f32
        ) + lbcat[:, g * 128:(g + 1) * 128]


@jax.jit
def _wprep(params):
    wcat = jnp.concatenate([params["conv_W_" + g] for g in "zrh"], axis=1)
    l1s = jnp.stack([params["lin_W_" + g][:F] for g in "zrh"])
    bcat = jnp.concatenate(
        [params["conv_b_" + g][None, :] for g in "zrh"], axis=1)
    lbcat = jnp.concatenate(
        [params["lin_b_" + g][None, :] for g in "zrh"], axis=1)
    wc, bc = pl.pallas_call(
        _wprep_body,
        out_shape=[jax.ShapeDtypeStruct((F, 3 * F), _f32),
                   jax.ShapeDtypeStruct((1, 3 * F), _f32)],
    )(wcat, l1s, bcat, lbcat)
    lq = jnp.concatenate([params["lin_W_z"][F:], params["lin_W_r"][F:]], axis=1)
    lh2 = params["lin_W_h"][F:]
    return wc, bc, lq, lh2


def _gru_body(zl0, zl1, zl2, zl3, zg0, zg1, zg2, zg3, dinvl, dinvg,
              wcl, bcl, lql, lh2l, wcg, bcg, lqg, lh2g, out):
    def run(zrefs, dinv_r, wc_r, bc_r, lq_r, lh2_r):
        wc = wc_r[...]
        bc = bc_r[...]
        lq = lq_r[...]
        lh2 = lh2_r[...]
        dinv = dinv_r[...]
        h = jnp.zeros((zrefs[0].shape[0], F), _f32)
        for t in range(T):
            st = zrefs[t][...] * dinv
            p = jnp.dot(st, wc, preferred_element_type=_f32) + bc
            q = jnp.dot(h, lq, preferred_element_type=_f32)
            z = jax.nn.sigmoid(p[:, :F] + q[:, :F])
            r = jax.nn.sigmoid(p[:, F:2 * F] + q[:, F:2 * F])
            ht = jnp.tanh(p[:, 2 * F:] +
                          jnp.dot(h * r, lh2, preferred_element_type=_f32))
            h = z * h + (1.0 - z) * ht
        return h
    out[:, :F] = run((zl0, zl1, zl2, zl3), dinvl, wcl, bcl, lql, lh2l)
    out[:, F:] = run((zg0, zg1, zg2, zg3), dinvg, wcg, bcg, lqg, lh2g)


@jax.jit
def _gru_kernel(z_list, dinvl, dinvg, wl, bl, lql, lh2l, wg, bg, lqg, lh2g):
    blk = 256
    grid = (NP // blk,)
    z_spec = pl.BlockSpec((blk, F), lambda i: (i, 0))
    d_spec = pl.BlockSpec((blk, 1), lambda i: (i, 0))
    full = lambda shape: pl.BlockSpec(shape, lambda i: (0, 0))
    return pl.pallas_call(
        _gru_body,
        grid=grid,
        in_specs=[z_spec] * 8 + [d_spec] * 2 + [
            full((F, 3 * F)), full((1, 3 * F)), full((F, 2 * F)), full((F, F)),
            full((F, 3 * F)), full((1, 3 * F)), full((F, 2 * F)), full((F, F)),
        ],
        out_specs=pl.BlockSpec((blk, 2 * F), lambda i: (i, 0)),
        out_shape=jax.ShapeDtypeStruct((NP, 2 * F), _f32),
    )(*z_list, dinvl, dinvg, wl, bl, lql, lh2l, wg, bg, lqg, lh2g)


def _prep_edges(ei, ew, epad):
    e = ew.shape[0]
    pad = epad - e
    fill = jnp.arange(pad, dtype=_i32) % N
    src = jnp.concatenate([ei[0].astype(_i32), fill]).reshape(-1, 128)
    dst = jnp.concatenate([ei[1].astype(_i32), fill]).reshape(-1, 128)
    w = jnp.concatenate([ew, jnp.zeros((pad,), _f32)]).reshape(-1, 128)
    return src, dst, w


def kernel(local_x, global_x, local_edge_index, global_edge_index,
           local_edge_weight, global_edge_weight, readout_batch,
           local_params, global_params):
    srcl, dstl, wl = _prep_edges(local_edge_index, local_edge_weight, EL_PAD)
    srcg, dstg, wg = _prep_edges(global_edge_index, global_edge_weight, EG_PAD)
    dinvl, dinvg = _deg_kernel(dstl, wl, dstg, wg)

    x = jnp.stack([
        jnp.pad(jnp.transpose(local_x, (2, 0, 1)), ((0, 0), (0, NP - N), (0, 0))),
        jnp.pad(jnp.transpose(global_x, (2, 0, 1)), ((0, 0), (0, NP - N), (0, 0))),
    ])
    dinv2 = jnp.stack([dinvl[:, None], dinvg[:, None]])
    y = _prescale(x, dinv2)

    z_list = _scatter_kernel(y, srcl, dstl, wl, srcg, dstg, wg)

    wcl, bcl, lql, lh2l = _wprep(local_params)
    wcg, bcg, lqg, lh2g = _wprep(global_params)
    out = _gru_kernel(z_list, dinvl[:, None], dinvg[:, None],
                      wcl, bcl, lql, lh2l, wcg, bcg, lqg, lh2g)
    return out[:N]


# speedup vs baseline: 33.1608x; 1.4779x over previous
import jax
import jax.numpy as jnp
from jax import lax
from jax.experimental import pallas as pl
from jax.experimental.pallas import tpu as pltpu
from jax.experimental.pallas import tpu_sc as plsc

N = 10000
NP = 10240
F = 128
T = 4
EL_PAD = 327680
EG_PAD = 163840
EW = 64
RPB = 32
NBLK_L = 10
NBLK_G = 5
SLICE = NP // 16

_f32 = jnp.float32
_i32 = jnp.int32


def _mesh():
    return plsc.VectorSubcoreMesh(core_axis_name="c", subcore_axis_name="s")


def _babylon_dinv(x):
    s = 0.5 * (1.0 + x)
    for _ in range(18):
        s = 0.5 * (s + x / s)
    return 1.0 / s


def _deg_body(dstl, wl, dstg, wg, dinvl, dinvg, buf_v, dst_b, w_b, spdeg):
    c = lax.axis_index("c")
    s = lax.axis_index("s")
    zero16 = jnp.zeros((16,), _f32)

    def run_graph(dst_h, w_h, out_h, nblk):
        def zbody(i, carry):
            buf_v[pl.ds(i * 16, 16)] = zero16
            return carry
        lax.fori_loop(0, NP // 16, zbody, 0)

        @pl.when(s == 0)
        def _():
            pltpu.sync_copy(buf_v, spdeg)
        plsc.subcore_barrier()

        base = s * nblk * RPB

        def sblk(blk, carry):
            r0 = base + blk * RPB
            pltpu.sync_copy(dst_h.at[pl.ds(r0, RPB)], dst_b)
            pltpu.sync_copy(w_h.at[pl.ds(r0, RPB)], w_b)
            for k in range(RPB):
                pltpu.sync_copy(w_b.at[k], spdeg.at[dst_b.at[k]], add=True)
            return carry
        lax.fori_loop(0, nblk, sblk, 0)
        plsc.subcore_barrier()

        pltpu.sync_copy(spdeg.at[pl.ds(s * SLICE, SLICE)],
                        buf_v.at[pl.ds(0, SLICE)])

        def dbody(i, carry):
            x = buf_v[pl.ds(i * 16, 16)] + 1.0
            buf_v[pl.ds(i * 16, 16)] = _babylon_dinv(x)
            return carry
        lax.fori_loop(0, SLICE // 16, dbody, 0)
        pltpu.sync_copy(buf_v.at[pl.ds(0, SLICE)],
                        out_h.at[pl.ds(s * SLICE, SLICE)])

    @pl.when(c == 0)
    def _():
        run_graph(dstl, wl, dinvl, NBLK_L)

    @pl.when(c == 1)
    def _():
        run_graph(dstg, wg, dinvg, NBLK_G)


@jax.jit
def _deg_kernel(dstl, wl, dstg, wg):
    out_type = [
        jax.ShapeDtypeStruct((NP,), _f32),
        jax.ShapeDtypeStruct((NP,), _f32),
    ]
    scratch = [
        pltpu.VMEM((NP,), _f32),
        pltpu.VMEM((RPB, EW), _i32),
        pltpu.VMEM((RPB, EW), _f32),
        pltpu.VMEM_SHARED((NP,), _f32),
    ]
    return pl.kernel(_deg_body, out_type=out_type, mesh=_mesh(),
                     scratch_types=scratch)(dstl, wl, dstg, wg)


def _prescale_body(x_ref, dinv_ref, y_ref):
    y_ref[...] = x_ref[...] * dinv_ref[...]


@jax.jit
def _prescale(x, dinv):
    blk = 1024
    return pl.pallas_call(
        _prescale_body,
        grid=(2, T, NP // blk),
        in_specs=[
            pl.BlockSpec((1, 1, blk, F), lambda g, t, i: (g, t, i, 0)),
            pl.BlockSpec((1, blk, 1), lambda g, t, i: (g, i, 0)),
        ],
        out_specs=pl.BlockSpec((1, 1, blk, F), lambda g, t, i: (g, t, i, 0)),
        out_shape=jax.ShapeDtypeStruct((2, T, NP, F), _f32),
    )(x, dinv)


def _scatter_pass(y4, src_h, dst_h, w_h, out4, q, oq, nblk, ebase, s,
                  gbuf, src_b, dst_b, w_b, accum, sg, ss):
    pltpu.sync_copy(y4.at[q, pl.ds(s * SLICE, SLICE)],
                    accum.at[pl.ds(s * SLICE, SLICE)])
    plsc.subcore_barrier()

    base = ebase + s * nblk * RPB
    y_h = y4.at[q]

    def scale(b, k):
        def gbody(g, carry):
            wv = w_b[k, pl.ds(g * 16, 16)]
            for i in range(16):
                wsc = lax.index_in_dim(wv, i, keepdims=False)
                e = g * 16 + i
                for j in range(8):
                    gbuf[b, e, pl.ds(j * 16, 16)] = (
                        gbuf[b, e, pl.ds(j * 16, 16)] * wsc)
            return carry
        lax.fori_loop(0, EW // 16, gbody, 0)

    def wait_gather(b):
        pltpu.make_async_copy(y_h.at[src_b.at[0]], gbuf.at[b], sg[b]).wait()

    def wait_scatter(b):
        pltpu.make_async_copy(
            gbuf.at[b], accum.at[dst_b.at[0]], ss[b]).wait()

    def bblk(blk, carry):
        r0 = base + blk * RPB
        pltpu.sync_copy(src_h.at[pl.ds(r0, RPB)], src_b)
        pltpu.sync_copy(dst_h.at[pl.ds(r0, RPB)], dst_b)
        pltpu.sync_copy(w_h.at[pl.ds(r0, RPB)], w_b)
        pltpu.async_copy(y_h.at[src_b.at[0]], gbuf.at[0], sg[0])
        pltpu.async_copy(y_h.at[src_b.at[1]], gbuf.at[1], sg[1])

        def round_(m, carry2):
            for b in range(4):
                k = m * 4 + b
                b2 = (b + 2) % 4
                wait_gather(b)
                scale(b, k)
                pltpu.async_copy(gbuf.at[b], accum.at[dst_b.at[k]],
                                 ss[b], add=True)
                if b < 2:
                    @pl.when(m > 0)
                    def _():
                        wait_scatter(b2)
                    pltpu.async_copy(y_h.at[src_b.at[k + 2]],
                                     gbuf.at[b2], sg[b2])
                else:
                    wait_scatter(b2)

                    @pl.when(m < RPB // 4 - 1)
                    def _():
                        pltpu.async_copy(y_h.at[src_b.at[k + 2]],
                                         gbuf.at[b2], sg[b2])
            return carry2
        lax.fori_loop(0, RPB // 4, round_, 0)
        wait_scatter(2)
        wait_scatter(3)
        return carry
    lax.fori_loop(0, nblk, bblk, 0)
    plsc.subcore_barrier()

    pltpu.sync_copy(accum.at[pl.ds(s * SLICE, SLICE)],
                    out4.at[oq, pl.ds(s * SLICE, SLICE)])


_LROWS = EL_PAD // EW


def _scatter_body(y4, srcA, dstA, wA, out4,
                  gbuf, src_b, dst_b, w_b, accum,
                  sg0, sg1, sg2, sg3, ss0, ss1, ss2, ss3):
    c = lax.axis_index("c")
    s = lax.axis_index("s")
    scr = (gbuf, src_b, dst_b, w_b, accum,
           (sg0, sg1, sg2, sg3), (ss0, ss1, ss2, ss3))

    def run_pass(q):
        is_local = q < 4
        ebase = jnp.where(is_local, 0, _LROWS)
        nblk = jnp.where(is_local, NBLK_L, NBLK_G)
        _scatter_pass(y4, srcA, dstA, wA, out4, q, q, nblk, ebase, s, *scr)
        return 0

    @pl.when(c == 0)
    def _():
        lax.fori_loop(0, 3, lambda p, car: run_pass(p), 0)

    @pl.when(c == 1)
    def _():
        lax.fori_loop(0, 5, lambda p, car: run_pass(jnp.where(p == 0, 3, p + 3)), 0)


@jax.jit
def _scatter_kernel(y4, srcA, dstA, wA):
    out_type = jax.ShapeDtypeStruct((8, NP, F), _f32)
    scratch = [
        pltpu.VMEM((4, EW, F), _f32),
        pltpu.VMEM((RPB, EW), _i32),
        pltpu.VMEM((RPB, EW), _i32),
        pltpu.VMEM((RPB, EW), _f32),
        pltpu.VMEM_SHARED((NP, F), _f32),
    ] + [pltpu.SemaphoreType.DMA] * 8
    return pl.kernel(_scatter_body, out_type=out_type, mesh=_mesh(),
                     scratch_types=scratch)(y4, srcA, dstA, wA)


def _wprep_body(wcat, l1s, bcat, lbcat, wc, bc):
    for g in range(3):
        l1 = l1s[g]
        wc[:, g * 128:(g + 1) * 128] = jnp.dot(
            wcat[:, g * 128:(g + 1) * 128], l1, preferred_element_type=_f32)
        bc[:, g * 128:(g + 1) * 128] = jnp.dot(
            bcat[:, g * 128:(g + 1) * 128], l1, preferred_element_type=_f32
        ) + lbcat[:, g * 128:(g + 1) * 128]


@jax.jit
def _wprep(params):
    wcat = jnp.concatenate([params["conv_W_" + g] for g in "zrh"], axis=1)
    l1s = jnp.stack([params["lin_W_" + g][:F] for g in "zrh"])
    bcat = jnp.concatenate(
        [params["conv_b_" + g][None, :] for g in "zrh"], axis=1)
    lbcat = jnp.concatenate(
        [params["lin_b_" + g][None, :] for g in "zrh"], axis=1)
    wc, bc = pl.pallas_call(
        _wprep_body,
        out_shape=[jax.ShapeDtypeStruct((F, 3 * F), _f32),
                   jax.ShapeDtypeStruct((1, 3 * F), _f32)],
    )(wcat, l1s, bcat, lbcat)
    lq = jnp.concatenate([params["lin_W_z"][F:], params["lin_W_r"][F:]], axis=1)
    lh2 = params["lin_W_h"][F:]
    return wc, bc, lq, lh2


def _gru_body(zref, dinvl, dinvg,
              wcl, bcl, lql, lh2l, wcg, bcg, lqg, lh2g, out):
    def run(zrefs, dinv_r, wc_r, bc_r, lq_r, lh2_r):
        wc = wc_r[...]
        bc = bc_r[...]
        lq = lq_r[...]
        lh2 = lh2_r[...]
        dinv = dinv_r[...]
        h = jnp.zeros((zrefs[0].shape[0], F), _f32)
        for t in range(T):
            st = zrefs[t][...] * dinv
            p = jnp.dot(st, wc, preferred_element_type=_f32) + bc
            q = jnp.dot(h, lq, preferred_element_type=_f32)
            z = jax.nn.sigmoid(p[:, :F] + q[:, :F])
            r = jax.nn.sigmoid(p[:, F:2 * F] + q[:, F:2 * F])
            ht = jnp.tanh(p[:, 2 * F:] +
                          jnp.dot(h * r, lh2, preferred_element_type=_f32))
            h = z * h + (1.0 - z) * ht
        return h
    zs = [zref.at[t] for t in range(8)]
    out[:, :F] = run(zs[:4], dinvl, wcl, bcl, lql, lh2l)
    out[:, F:] = run(zs[4:], dinvg, wcg, bcg, lqg, lh2g)


@jax.jit
def _gru_kernel(z8, dinvl, dinvg, wl, bl, lql, lh2l, wg, bg, lqg, lh2g):
    blk = 256
    grid = (NP // blk,)
    z_spec = pl.BlockSpec((8, blk, F), lambda i: (0, i, 0))
    d_spec = pl.BlockSpec((blk, 1), lambda i: (i, 0))
    full = lambda shape: pl.BlockSpec(shape, lambda i: (0, 0))
    return pl.pallas_call(
        _gru_body,
        grid=grid,
        in_specs=[z_spec] + [d_spec] * 2 + [
            full((F, 3 * F)), full((1, 3 * F)), full((F, 2 * F)), full((F, F)),
            full((F, 3 * F)), full((1, 3 * F)), full((F, 2 * F)), full((F, F)),
        ],
        out_specs=pl.BlockSpec((blk, 2 * F), lambda i: (i, 0)),
        out_shape=jax.ShapeDtypeStruct((NP, 2 * F), _f32),
    )(z8, dinvl, dinvg, wl, bl, lql, lh2l, wg, bg, lqg, lh2g)


def _prep_edges(ei, ew, epad):
    e = ew.shape[0]
    pad = epad - e
    fill = jnp.arange(pad, dtype=_i32) % N
    src = jnp.concatenate([ei[0].astype(_i32), fill]).reshape(-1, EW)
    dst = jnp.concatenate([ei[1].astype(_i32), fill]).reshape(-1, EW)
    w = jnp.concatenate([ew, jnp.zeros((pad,), _f32)]).reshape(-1, EW)
    return src, dst, w


def kernel(local_x, global_x, local_edge_index, global_edge_index,
           local_edge_weight, global_edge_weight, readout_batch,
           local_params, global_params):
    srcl, dstl, wl = _prep_edges(local_edge_index, local_edge_weight, EL_PAD)
    srcg, dstg, wg = _prep_edges(global_edge_index, global_edge_weight, EG_PAD)
    dinvl, dinvg = _deg_kernel(dstl, wl, dstg, wg)

    x = jnp.stack([
        jnp.pad(jnp.transpose(local_x, (2, 0, 1)), ((0, 0), (0, NP - N), (0, 0))),
        jnp.pad(jnp.transpose(global_x, (2, 0, 1)), ((0, 0), (0, NP - N), (0, 0))),
    ])
    dinv2 = jnp.stack([dinvl[:, None], dinvg[:, None]])
    y4 = _prescale(x, dinv2).reshape(8, NP, F)

    srcA = jnp.concatenate([srcl, srcg])
    dstA = jnp.concatenate([dstl, dstg])
    wA = jnp.concatenate([wl, wg])
    z8 = _scatter_kernel(y4, srcA, dstA, wA)

    wcl, bcl, lql, lh2l = _wprep(local_params)
    wcg, bcg, lqg, lh2g = _wprep(global_params)
    out = _gru_kernel(z8, dinvl[:, None], dinvg[:, None],
                      wcl, bcl, lql, lh2l, wcg, bcg, lqg, lh2g)
    return out[:N]


# speedup vs baseline: 33.8859x; 1.0219x over previous
import jax
import jax.numpy as jnp
from jax import lax
from jax.experimental import pallas as pl
from jax.experimental.pallas import tpu as pltpu
from jax.experimental.pallas import tpu_sc as plsc

N = 10000
NP = 10240
F = 128
T = 4
EL_PAD = 327680
EG_PAD = 163840
EW = 64
RPB = 32
NBLK_L = 10
NBLK_G = 5
SLICE = NP // 16

_f32 = jnp.float32
_i32 = jnp.int32


def _mesh():
    return plsc.VectorSubcoreMesh(core_axis_name="c", subcore_axis_name="s")


def _babylon_dinv(x):
    s = 0.5 * (1.0 + x)
    for _ in range(18):
        s = 0.5 * (s + x / s)
    return 1.0 / s


def _deg_body(dstl, wl, dstg, wg, dinvl, dinvg, buf_v, dst_b, w_b, spdeg):
    c = lax.axis_index("c")
    s = lax.axis_index("s")
    zero16 = jnp.zeros((16,), _f32)

    def run_graph(dst_h, w_h, out_h, nblk):
        def zbody(i, carry):
            buf_v[pl.ds(i * 16, 16)] = zero16
            return carry
        lax.fori_loop(0, NP // 16, zbody, 0)

        @pl.when(s == 0)
        def _():
            pltpu.sync_copy(buf_v, spdeg)
        plsc.subcore_barrier()

        base = s * nblk * RPB

        def sblk(blk, carry):
            r0 = base + blk * RPB
            pltpu.sync_copy(dst_h.at[pl.ds(r0, RPB)], dst_b)
            pltpu.sync_copy(w_h.at[pl.ds(r0, RPB)], w_b)
            for k in range(RPB):
                pltpu.sync_copy(w_b.at[k], spdeg.at[dst_b.at[k]], add=True)
            return carry
        lax.fori_loop(0, nblk, sblk, 0)
        plsc.subcore_barrier()

        pltpu.sync_copy(spdeg.at[pl.ds(s * SLICE, SLICE)],
                        buf_v.at[pl.ds(0, SLICE)])

        def dbody(i, carry):
            x = buf_v[pl.ds(i * 16, 16)] + 1.0
            buf_v[pl.ds(i * 16, 16)] = _babylon_dinv(x)
            return carry
        lax.fori_loop(0, SLICE // 16, dbody, 0)
        pltpu.sync_copy(buf_v.at[pl.ds(0, SLICE)],
                        out_h.at[pl.ds(s * SLICE, SLICE)])

    @pl.when(c == 0)
    def _():
        run_graph(dstl, wl, dinvl, NBLK_L)

    @pl.when(c == 1)
    def _():
        run_graph(dstg, wg, dinvg, NBLK_G)


@jax.jit
def _deg_kernel(dstl, wl, dstg, wg):
    out_type = [
        jax.ShapeDtypeStruct((NP,), _f32),
        jax.ShapeDtypeStruct((NP,), _f32),
    ]
    scratch = [
        pltpu.VMEM((NP,), _f32),
        pltpu.VMEM((RPB, EW), _i32),
        pltpu.VMEM((RPB, EW), _f32),
        pltpu.VMEM_SHARED((NP,), _f32),
    ]
    return pl.kernel(_deg_body, out_type=out_type, mesh=_mesh(),
                     scratch_types=scratch)(dstl, wl, dstg, wg)


def _prescale_body(x_ref, dinv_ref, y_ref):
    y_ref[...] = x_ref[...] * dinv_ref[...]


@jax.jit
def _prescale(x, dinv):
    blk = 1024
    return pl.pallas_call(
        _prescale_body,
        grid=(2, T, NP // blk),
        in_specs=[
            pl.BlockSpec((1, 1, blk, F), lambda g, t, i: (g, t, i, 0)),
            pl.BlockSpec((1, blk, 1), lambda g, t, i: (g, i, 0)),
        ],
        out_specs=pl.BlockSpec((1, 1, blk, F), lambda g, t, i: (g, t, i, 0)),
        out_shape=jax.ShapeDtypeStruct((2, T, NP, F), _f32),
    )(x, dinv)


def _scatter_pass(y4, src_h, dst_h, w_h, out4, q, oq, nblk, ebase, s,
                  gbuf, src_b, dst_b, w_b, accum, sg, ss):
    pltpu.sync_copy(y4.at[q, pl.ds(s * SLICE, SLICE)],
                    accum.at[pl.ds(s * SLICE, SLICE)])
    plsc.subcore_barrier()

    base = ebase + s * nblk * RPB
    y_h = y4.at[q]

    def scale(b, k):
        def gbody(g, carry):
            wv = w_b[k, pl.ds(g * 16, 16)]
            for i in range(16):
                wsc = lax.index_in_dim(wv, i, keepdims=False)
                e = g * 16 + i
                for j in range(8):
                    gbuf[b, e, pl.ds(j * 16, 16)] = (
                        gbuf[b, e, pl.ds(j * 16, 16)] * wsc)
            return carry
        lax.fori_loop(0, EW // 16, gbody, 0)

    def wait_gather(b):
        pltpu.make_async_copy(y_h.at[src_b.at[0]], gbuf.at[b], sg[b]).wait()

    def wait_scatter(b):
        pltpu.make_async_copy(
            gbuf.at[b], accum.at[dst_b.at[0]], ss[b]).wait()

    def bblk(blk, carry):
        r0 = base + blk * RPB
        pltpu.sync_copy(src_h.at[pl.ds(r0, RPB)], src_b)
        pltpu.sync_copy(dst_h.at[pl.ds(r0, RPB)], dst_b)
        pltpu.sync_copy(w_h.at[pl.ds(r0, RPB)], w_b)
        pltpu.async_copy(y_h.at[src_b.at[0]], gbuf.at[0], sg[0])
        pltpu.async_copy(y_h.at[src_b.at[1]], gbuf.at[1], sg[1])

        def round_(m, carry2):
            for b in range(4):
                k = m * 4 + b
                b2 = (b + 2) % 4
                wait_gather(b)
                scale(b, k)
                pltpu.async_copy(gbuf.at[b], accum.at[dst_b.at[k]],
                                 ss[b], add=True)
                if b < 2:
                    @pl.when(m > 0)
                    def _():
                        wait_scatter(b2)
                    pltpu.async_copy(y_h.at[src_b.at[k + 2]],
                                     gbuf.at[b2], sg[b2])
                else:
                    wait_scatter(b2)

                    @pl.when(m < RPB // 4 - 1)
                    def _():
                        pltpu.async_copy(y_h.at[src_b.at[k + 2]],
                                         gbuf.at[b2], sg[b2])
            return carry2
        lax.fori_loop(0, RPB // 4, round_, 0)
        wait_scatter(2)
        wait_scatter(3)
        return carry
    lax.fori_loop(0, nblk, bblk, 0)
    plsc.subcore_barrier()

    pltpu.sync_copy(accum.at[pl.ds(s * SLICE, SLICE)],
                    out4.at[oq, pl.ds(s * SLICE, SLICE)])


_LROWS = EL_PAD // EW


def _scatter_body(y4, srcA, dstA, wA, out4,
                  gbuf, src_b, dst_b, w_b, accum,
                  sg0, sg1, sg2, sg3, ss0, ss1, ss2, ss3):
    c = lax.axis_index("c")
    s = lax.axis_index("s")
    scr = (gbuf, src_b, dst_b, w_b, accum,
           (sg0, sg1, sg2, sg3), (ss0, ss1, ss2, ss3))

    def run_pass(q):
        is_local = q < 4
        ebase = jnp.where(is_local, 0, _LROWS)
        nblk = jnp.where(is_local, NBLK_L, NBLK_G)
        _scatter_pass(y4, srcA, dstA, wA, out4, q, q, nblk, ebase, s, *scr)
        return 0

    @pl.when(c == 0)
    def _():
        lax.fori_loop(0, 3, lambda p, car: run_pass(p), 0)

    @pl.when(c == 1)
    def _():
        lax.fori_loop(0, 5, lambda p, car: run_pass(jnp.where(p == 0, 3, p + 3)), 0)


@jax.jit
def _scatter_kernel(y4, srcA, dstA, wA):
    out_type = jax.ShapeDtypeStruct((8, NP, F), _f32)
    scratch = [
        pltpu.VMEM((4, EW, F), _f32),
        pltpu.VMEM((RPB, EW), _i32),
        pltpu.VMEM((RPB, EW), _i32),
        pltpu.VMEM((RPB, EW), _f32),
        pltpu.VMEM_SHARED((NP, F), _f32),
    ] + [pltpu.SemaphoreType.DMA] * 8
    return pl.kernel(_scatter_body, out_type=out_type, mesh=_mesh(),
                     scratch_types=scratch)(y4, srcA, dstA, wA)


def _wprep_body(wcat, l1s, bcat, lbcat, wc, bc):
    for g in range(3):
        l1 = l1s[g]
        wc[:, g * 128:(g + 1) * 128] = jnp.dot(
            wcat[:, g * 128:(g + 1) * 128], l1, preferred_element_type=_f32)
        bc[:, g * 128:(g + 1) * 128] = jnp.dot(
            bcat[:, g * 128:(g + 1) * 128], l1, preferred_element_type=_f32
        ) + lbcat[:, g * 128:(g + 1) * 128]


@jax.jit
def _wprep(params):
    wcat = jnp.concatenate([params["conv_W_" + g] for g in "zrh"], axis=1)
    l1s = jnp.stack([params["lin_W_" + g][:F] for g in "zrh"])
    bcat = jnp.concatenate(
        [params["conv_b_" + g][None, :] for g in "zrh"], axis=1)
    lbcat = jnp.concatenate(
        [params["lin_b_" + g][None, :] for g in "zrh"], axis=1)
    wc, bc = pl.pallas_call(
        _wprep_body,
        out_shape=[jax.ShapeDtypeStruct((F, 3 * F), _f32),
                   jax.ShapeDtypeStruct((1, 3 * F), _f32)],
    )(wcat, l1s, bcat, lbcat)
    lq = jnp.concatenate([params["lin_W_z"][F:], params["lin_W_r"][F:]], axis=1)
    lh2 = params["lin_W_h"][F:]
    return wc, bc, lq, lh2


def _gru_body(zref, dinvl, dinvg,
              wcl, bcl, lql, lh2l, wcg, bcg, lqg, lh2g, out):
    def run(zrefs, dinv_r, wc_r, bc_r, lq_r, lh2_r):
        wc = wc_r[...]
        bc = bc_r[...]
        lq = lq_r[...]
        lh2 = lh2_r[...]
        dinv = dinv_r[...]
        h = jnp.zeros((zrefs[0].shape[0], F), _f32)
        for t in range(T):
            st = zrefs[t][...] * dinv
            p = jnp.dot(st, wc, preferred_element_type=_f32) + bc
            q = jnp.dot(h, lq, preferred_element_type=_f32)
            z = jax.nn.sigmoid(p[:, :F] + q[:, :F])
            r = jax.nn.sigmoid(p[:, F:2 * F] + q[:, F:2 * F])
            ht = jnp.tanh(p[:, 2 * F:] +
                          jnp.dot(h * r, lh2, preferred_element_type=_f32))
            h = z * h + (1.0 - z) * ht
        return h
    zs = [zref.at[t] for t in range(8)]
    out[:, :F] = run(zs[:4], dinvl, wcl, bcl, lql, lh2l)
    out[:, F:] = run(zs[4:], dinvg, wcg, bcg, lqg, lh2g)


@jax.jit
def _gru_kernel(z8, dinvl, dinvg, wl, bl, lql, lh2l, wg, bg, lqg, lh2g):
    blk = 512
    grid = (NP // blk,)
    z_spec = pl.BlockSpec((8, blk, F), lambda i: (0, i, 0))
    d_spec = pl.BlockSpec((blk, 1), lambda i: (i, 0))
    full = lambda shape: pl.BlockSpec(shape, lambda i: (0, 0))
    return pl.pallas_call(
        _gru_body,
        grid=grid,
        in_specs=[z_spec] + [d_spec] * 2 + [
            full((F, 3 * F)), full((1, 3 * F)), full((F, 2 * F)), full((F, F)),
            full((F, 3 * F)), full((1, 3 * F)), full((F, 2 * F)), full((F, F)),
        ],
        out_specs=pl.BlockSpec((blk, 2 * F), lambda i: (i, 0)),
        out_shape=jax.ShapeDtypeStruct((NP, 2 * F), _f32),
    )(z8, dinvl, dinvg, wl, bl, lql, lh2l, wg, bg, lqg, lh2g)


def _prep_edges(ei, ew, epad):
    e = ew.shape[0]
    pad = epad - e
    fill = jnp.arange(pad, dtype=_i32) % N
    src = jnp.concatenate([ei[0].astype(_i32), fill]).reshape(-1, EW)
    dst = jnp.concatenate([ei[1].astype(_i32), fill]).reshape(-1, EW)
    w = jnp.concatenate([ew, jnp.zeros((pad,), _f32)]).reshape(-1, EW)
    return src, dst, w


def kernel(local_x, global_x, local_edge_index, global_edge_index,
           local_edge_weight, global_edge_weight, readout_batch,
           local_params, global_params):
    srcl, dstl, wl = _prep_edges(local_edge_index, local_edge_weight, EL_PAD)
    srcg, dstg, wg = _prep_edges(global_edge_index, global_edge_weight, EG_PAD)
    dinvl, dinvg = _deg_kernel(dstl, wl, dstg, wg)

    x = jnp.stack([
        jnp.pad(jnp.transpose(local_x, (2, 0, 1)), ((0, 0), (0, NP - N), (0, 0))),
        jnp.pad(jnp.transpose(global_x, (2, 0, 1)), ((0, 0), (0, NP - N), (0, 0))),
    ])
    dinv2 = jnp.stack([dinvl[:, None], dinvg[:, None]])
    y4 = _prescale(x, dinv2).reshape(8, NP, F)

    srcA = jnp.concatenate([srcl, srcg])
    dstA = jnp.concatenate([dstl, dstg])
    wA = jnp.concatenate([wl, wg])
    z8 = _scatter_kernel(y4, srcA, dstA, wA)

    wcl, bcl, lql, lh2l = _wprep(local_params)
    wcg, bcg, lqg, lh2g = _wprep(global_params)
    out = _gru_kernel(z8, dinvl[:, None], dinvg[:, None],
                      wcl, bcl, lql, lh2l, wcg, bcg, lqg, lh2g)
    return out[:N]
